# Initial kernel scaffold; baseline (speedup 1.0000x reference)
import jax
import jax.numpy as jnp
from jax.experimental import pallas as pl


def kernel(input, temperature, norm_w, norm_b, qkv_w, dwconv_w, out_w, out_b, a1, a2, a3, a4):
    raise NotImplementedError("write your pallas kernel here")


# speedup vs baseline: 1.3286x; 1.3286x over previous
import jax
import jax.numpy as jnp
import numpy as np
from jax.experimental import pallas as pl

N_HEAD = 4
NORM_GROUPS = 32
IN_CH = 192
C_HEAD = IN_CH // N_HEAD
L = 64 * 64
TOPKS = (C_HEAD // 2, C_HEAD * 2 // 3, C_HEAD * 3 // 4, C_HEAD * 4 // 5)

_HI = jax.lax.Precision.HIGHEST

_IND = (np.arange(IN_CH)[:, None] // (IN_CH // NORM_GROUPS)
        == np.arange(NORM_GROUPS)[None, :]).astype(np.float32)


def _prep_kernel(x_ref, qkvw_ref, dww_ref, nw_ref, nb_ref, ind_ref,
                 q_ref, k_ref, v_ref):
    x = x_ref[0]
    npix = jnp.float32((IN_CH // NORM_GROUPS) * L)
    s1 = jnp.sum(x, axis=1, keepdims=True)
    s2 = jnp.sum(x * x, axis=1, keepdims=True)
    ind = ind_ref[...]
    gs1 = jnp.dot(ind.T, s1, precision=_HI)
    gs2 = jnp.dot(ind.T, s2, precision=_HI)
    mean = jnp.dot(ind, gs1, precision=_HI) / npix
    ex2 = jnp.dot(ind, gs2, precision=_HI) / npix
    var = ex2 - mean * mean
    xn = (x - mean) * jax.lax.rsqrt(var + 1e-5)
    xn = xn * nw_ref[...] + nb_ref[...]
    xnb = xn.astype(jnp.bfloat16)
    r = jax.lax.broadcasted_iota(jnp.int32, (1, L), 1)
    wpos = r % 64
    hpos = r // 64
    for c, dst in enumerate((q_ref, k_ref, v_ref)):
        cs = slice(c * IN_CH, (c + 1) * IN_CH)
        qkv = jnp.dot(qkvw_ref[cs].astype(jnp.bfloat16), xnb,
                      preferred_element_type=jnp.float32)
        qkvb = qkv.astype(jnp.bfloat16).astype(jnp.float32)
        dwb = dww_ref[cs]
        y = jnp.zeros_like(qkv)
        for dh in (-1, 0, 1):
            for dw in (-1, 0, 1):
                t = 3 * (dh + 1) + (dw + 1)
                sh = (dh * 64 + dw) % L
                shifted = qkvb if sh == 0 else jnp.concatenate(
                    [qkvb[:, sh:], qkvb[:, :sh]], axis=1)
                valid = ((wpos + dw >= 0) & (wpos + dw < 64)
                         & (hpos + dh >= 0) & (hpos + dh < 64))
                y = y + jnp.where(valid, shifted, 0.0) * dwb[:, t:t + 1]
        if c < 2:
            n = jnp.sqrt(jnp.sum(y * y, axis=1, keepdims=True))
            y = y / jnp.maximum(n, 1e-12)
        dst[0] = y


def _attn_kernel(attn_ref, v_ref, avec_ref, o_ref):
    attn = attn_ref[0, 0]
    col = jax.lax.broadcasted_iota(jnp.int32, (C_HEAD, C_HEAD), 1)
    row = jax.lax.broadcasted_iota(jnp.int32, (C_HEAD, C_HEAD), 0)
    a_i = attn[:, None, :]
    a_l = attn[:, :, None]
    beats = (a_l > a_i) | ((a_l == a_i) & (row < col)[None, :, :])
    rank = jnp.sum(beats.astype(jnp.float32), axis=1)
    e = jnp.exp(attn - jnp.max(attn, axis=1, keepdims=True))
    w = jnp.zeros_like(attn)
    for i, kk in enumerate(TOPKS):
        m = (rank < kk).astype(jnp.float32)
        s = jnp.sum(e * m, axis=1, keepdims=True)
        w = w + avec_ref[i, 0] * m * e / s
    o_ref[0, 0] = jnp.dot(w, v_ref[0, 0], precision=_HI,
                          preferred_element_type=jnp.float32)


def _outconv_kernel(x_ref, outw_ref, outb_ref, o_ref):
    o_ref[0] = jnp.dot(outw_ref[...], x_ref[0], precision=_HI,
                       preferred_element_type=jnp.float32) + outb_ref[...]


def kernel(input, temperature, norm_w, norm_b, qkv_w, dwconv_w, out_w, out_b,
           a1, a2, a3, a4):
    batch = input.shape[0]
    x = input.reshape(batch, IN_CH, L).astype(jnp.float32)
    qkvw = qkv_w.reshape(3 * IN_CH, IN_CH)
    dww = dwconv_w.reshape(3 * IN_CH, 9)
    nw = norm_w.reshape(IN_CH, 1)
    nb = norm_b.reshape(IN_CH, 1)

    q, k, v = pl.pallas_call(
        _prep_kernel,
        grid=(batch,),
        in_specs=[
            pl.BlockSpec((1, IN_CH, L), lambda b: (b, 0, 0)),
            pl.BlockSpec((3 * IN_CH, IN_CH), lambda b: (0, 0)),
            pl.BlockSpec((3 * IN_CH, 9), lambda b: (0, 0)),
            pl.BlockSpec((IN_CH, 1), lambda b: (0, 0)),
            pl.BlockSpec((IN_CH, 1), lambda b: (0, 0)),
            pl.BlockSpec((IN_CH, NORM_GROUPS), lambda b: (0, 0)),
        ],
        out_specs=[pl.BlockSpec((1, IN_CH, L), lambda b: (b, 0, 0))] * 3,
        out_shape=[jax.ShapeDtypeStruct((batch, IN_CH, L), jnp.float32)] * 3,
    )(x, qkvw, dww, nw, nb, jnp.asarray(_IND))

    q4 = q.reshape(batch, N_HEAD, C_HEAD, L)
    k4 = k.reshape(batch, N_HEAD, C_HEAD, L)
    q_fft = jnp.fft.fftshift(jnp.fft.rfft2(q4))
    k_fft = jnp.fft.fftshift(jnp.fft.rfft2(k4))
    attn = (q_fft @ jnp.swapaxes(k_fft, -2, -1)) * temperature
    attn = jnp.fft.ifftshift(attn)
    attn = jnp.fft.irfft2(attn, s=(C_HEAD, C_HEAD))

    avec = jnp.stack([a1[0], a2[0], a3[0], a4[0]]).reshape(4, 1)
    v4 = v.reshape(batch, N_HEAD, C_HEAD, L)

    hout = pl.pallas_call(
        _attn_kernel,
        grid=(batch, N_HEAD),
        in_specs=[
            pl.BlockSpec((1, 1, C_HEAD, C_HEAD), lambda b, h: (b, h, 0, 0)),
            pl.BlockSpec((1, 1, C_HEAD, L), lambda b, h: (b, h, 0, 0)),
            pl.BlockSpec((4, 1), lambda b, h: (0, 0)),
        ],
        out_specs=pl.BlockSpec((1, 1, C_HEAD, L), lambda b, h: (b, h, 0, 0)),
        out_shape=jax.ShapeDtypeStruct((batch, N_HEAD, C_HEAD, L),
                                       jnp.float32),
    )(attn, v4, avec)

    out = pl.pallas_call(
        _outconv_kernel,
        grid=(batch,),
        in_specs=[
            pl.BlockSpec((1, IN_CH, L), lambda b: (b, 0, 0)),
            pl.BlockSpec((IN_CH, IN_CH), lambda b: (0, 0)),
            pl.BlockSpec((IN_CH, 1), lambda b: (0, 0)),
        ],
        out_specs=pl.BlockSpec((1, IN_CH, L), lambda b: (b, 0, 0)),
        out_shape=jax.ShapeDtypeStruct((batch, IN_CH, L), jnp.float32),
    )(hout.reshape(batch, IN_CH, L), out_w.reshape(IN_CH, IN_CH),
      out_b.reshape(IN_CH, 1))

    return out.reshape(batch, IN_CH, 64, 64)
